# baseline (speedup 1.0000x reference)
import functools

import jax
import jax.numpy as jnp
from jax import lax
from jax.experimental import pallas as pl
from jax.experimental.pallas import tpu as pltpu

_N = 32768
_K = 1024
_D = 64
_BN = 4096
_GRID = _N // _BN


def _vq_body(x_ref, xsq_ref, cbt2_ref, cbsq_ref, cb_ref,
             codes_ref, quant_ref, loss_ref):
    i = pl.program_id(0)

    x = x_ref[...].reshape(_BN, _D)
    xsq = xsq_ref[...]

    ch = 128
    colf = jax.lax.broadcasted_iota(jnp.int32, (1, ch), 1).astype(jnp.float32)
    bv = None
    bi = None
    for c in range(_K // ch):
        dot2_c = jnp.dot(x, cbt2_ref[:, c * ch:(c + 1) * ch],
                         preferred_element_type=jnp.float32)
        dc = (xsq - dot2_c) + cbsq_ref[:, c * ch:(c + 1) * ch]
        if c == 0:
            bv = dc
            bi = jnp.broadcast_to(colf, (_BN, ch))
        else:
            m = dc < bv
            bi = jnp.where(m, colf + jnp.float32(c * ch), bi)
            bv = jnp.minimum(bv, dc)
    minval = jnp.min(bv, axis=-1, keepdims=True)
    bim = jnp.where(bv == minval, bi, jnp.float32(_K))
    codef = jnp.min(bim, axis=-1, keepdims=True)

    code = codef.astype(jnp.int32)
    codes_ref[...] = code.reshape(1, _BN // 128, 128)
    colf_full = jax.lax.broadcasted_iota(
        jnp.int32, (1, _K), 1).astype(jnp.float32)
    onehot = (colf_full == codef).astype(jnp.bfloat16)
    q = jnp.dot(onehot, cb_ref[...], preferred_element_type=jnp.float32)

    d = q - x
    quant_ref[...] = (x + d).astype(jnp.bfloat16).reshape(1, _BN, _D)

    part = jnp.sum(d * d)

    @pl.when(i == 0)
    def _():
        loss_ref[0, 0] = 0.0

    loss_ref[0, 0] += part


@jax.jit
def kernel(x, codebook):
    x_flat = x.reshape(-1, _D).astype(jnp.float32)
    cb = codebook.astype(jnp.float32)
    x_sq = jnp.sum(x_flat ** 2, axis=-1, keepdims=True)
    cb_sq = jnp.sum(cb ** 2, axis=-1).reshape(1, _K)
    cbt2 = (cb + cb).T

    codes3, quant, loss_sum = pl.pallas_call(
        _vq_body,
        grid=(_GRID,),
        in_specs=[
            pl.BlockSpec((1, _BN, _D), lambda i: (i, 0, 0)),
            pl.BlockSpec((_BN, 1), lambda i: (i, 0)),
            pl.BlockSpec((_D, _K), lambda i: (0, 0)),
            pl.BlockSpec((1, _K), lambda i: (0, 0)),
            pl.BlockSpec((_K, _D), lambda i: (0, 0)),
        ],
        out_specs=[
            pl.BlockSpec((1, _BN // 128, 128), lambda i: (i, 0, 0)),
            pl.BlockSpec((1, _BN, _D), lambda i: (i, 0, 0)),
            pl.BlockSpec(memory_space=pltpu.SMEM, block_shape=(1, 1),
                         index_map=lambda i: (0, 0)),
        ],
        out_shape=[
            jax.ShapeDtypeStruct((_GRID, _BN // 128, 128), jnp.int32),
            jax.ShapeDtypeStruct((_GRID, _BN, _D), jnp.bfloat16),
            jax.ShapeDtypeStruct((1, 1), jnp.float32),
        ],
    )(x.reshape(_GRID, _BN, _D).astype(jnp.float32), x_sq, cbt2, cb_sq,
      cb.astype(jnp.bfloat16))

    loss = loss_sum[0, 0] / jnp.float32(_N * _D)
    quantized = quant.reshape(x.shape)
    codes_out = codes3.reshape(x.shape[:-1])
    return (quantized, codes_out, loss, loss)


# speedup vs baseline: 1.0273x; 1.0273x over previous
import functools

import jax
import jax.numpy as jnp
from jax import lax
from jax.experimental import pallas as pl
from jax.experimental.pallas import tpu as pltpu

_N = 32768
_K = 1024
_D = 64
_BN = 4096
_GRID = _N // _BN


def _vq_body(x_ref, xsq_ref, cbt2_ref, cbsq_ref, cb_ref,
             codes_ref, quant_ref, loss_ref):
    i = pl.program_id(0)

    x = x_ref[...].reshape(_BN, _D)
    xsq = xsq_ref[...]

    ch = 128
    colf = jax.lax.broadcasted_iota(jnp.int32, (1, ch), 1).astype(jnp.float32)
    bv = None
    bi = None
    for c in range(_K // ch):
        dot2_c = jnp.dot(x, cbt2_ref[:, c * ch:(c + 1) * ch],
                         preferred_element_type=jnp.float32)
        dc = (xsq - dot2_c) + cbsq_ref[:, c * ch:(c + 1) * ch]
        if c == 0:
            bv = dc
            bi = jnp.broadcast_to(colf, (_BN, ch))
        else:
            m = dc < bv
            bi = jnp.where(m, colf + jnp.float32(c * ch), bi)
            bv = jnp.minimum(bv, dc)
    minval = jnp.min(bv, axis=-1, keepdims=True)
    bim = jnp.where(bv == minval, bi, jnp.float32(_K))
    codef = jnp.min(bim, axis=-1, keepdims=True)

    code = codef.astype(jnp.int32)
    codes_ref[...] = code.reshape(1, 1, _BN)
    colf_full = jax.lax.broadcasted_iota(
        jnp.int32, (1, _K), 1).astype(jnp.float32)
    onehot = (colf_full == codef).astype(jnp.bfloat16)
    q = jnp.dot(onehot, cb_ref[...], preferred_element_type=jnp.float32)

    d = q - x
    quant_ref[...] = (x + d).astype(jnp.bfloat16).reshape(1, _BN, _D)

    part = jnp.sum(d * d)

    @pl.when(i == 0)
    def _():
        loss_ref[0, 0] = 0.0

    loss_ref[0, 0] += part


@jax.jit
def kernel(x, codebook):
    x_flat = x.reshape(-1, _D).astype(jnp.float32)
    cb = codebook.astype(jnp.float32)
    x_sq = jnp.sum(x_flat ** 2, axis=-1, keepdims=True)
    cb_sq = jnp.sum(cb ** 2, axis=-1).reshape(1, _K)
    cbt2 = (cb + cb).T

    codes3, quant, loss_sum = pl.pallas_call(
        _vq_body,
        grid=(_GRID,),
        in_specs=[
            pl.BlockSpec((1, _BN, _D), lambda i: (i, 0, 0)),
            pl.BlockSpec((_BN, 1), lambda i: (i, 0)),
            pl.BlockSpec((_D, _K), lambda i: (0, 0)),
            pl.BlockSpec((1, _K), lambda i: (0, 0)),
            pl.BlockSpec((_K, _D), lambda i: (0, 0)),
        ],
        out_specs=[
            pl.BlockSpec((1, 1, _BN), lambda i: (i, 0, 0)),
            pl.BlockSpec((1, _BN, _D), lambda i: (i, 0, 0)),
            pl.BlockSpec(memory_space=pltpu.SMEM, block_shape=(1, 1),
                         index_map=lambda i: (0, 0)),
        ],
        out_shape=[
            jax.ShapeDtypeStruct((_GRID, 1, _BN), jnp.int32),
            jax.ShapeDtypeStruct((_GRID, _BN, _D), jnp.bfloat16),
            jax.ShapeDtypeStruct((1, 1), jnp.float32),
        ],
    )(x.reshape(_GRID, _BN, _D).astype(jnp.float32), x_sq, cbt2, cb_sq,
      cb.astype(jnp.bfloat16))

    loss = loss_sum[0, 0] / jnp.float32(_N * _D)
    quantized = quant.reshape(x.shape)
    codes_out = codes3.reshape(x.shape[:-1])
    return (quantized, codes_out, loss, loss)
